# baseline (speedup 1.0000x reference)
import functools

import jax
import jax.numpy as jnp
from jax.experimental import pallas as pl
from jax.experimental.pallas import tpu as pltpu

B, N, D, H, F = 4, 1024, 128, 8, 64
R = 256

_NEG_SLOPE = 0.2
_HI = jax.lax.Precision.HIGHEST


def _vec_kernel(x_ref, W_ref, asrc_ref, adst_ref,
                E1_ref, E2_ref, F1_ref, F2_ref):
    xb = x_ref[0]
    W = W_ref[...]
    u_src = jax.lax.dot_general(
        W, asrc_ref[...], (((2,), (1,)), ((0,), (0,))),
        precision=_HI, preferred_element_type=jnp.float32)
    u_dst = jax.lax.dot_general(
        W, adst_ref[...], (((2,), (1,)), ((0,), (0,))),
        precision=_HI, preferred_element_type=jnp.float32)
    s = jax.lax.dot_general(
        xb, u_src, (((1,), (1,)), ((), ())),
        precision=_HI, preferred_element_type=jnp.float32)
    dT = jax.lax.dot_general(
        u_dst, xb, (((1,), (1,)), ((), ())),
        precision=_HI, preferred_element_type=jnp.float32)
    dmax = jnp.max(dT, axis=1)
    sm = s + dmax[None, :]
    m = jnp.maximum(sm, _NEG_SLOPE * sm)
    E1_ref[0] = jnp.exp(s - m)
    E2_ref[0] = jnp.exp(_NEG_SLOPE * s - m)
    F1_ref[0] = jnp.exp(dT)
    F2_ref[0] = jnp.exp(_NEG_SLOPE * dT)


def _adj_kernel(Ep_ref, Fhat_ref, Fc_ref, out_ref):
    Ep = Ep_ref[0]
    Fhat = Fhat_ref[0]
    Fc = Fc_ref[0]

    Q = jax.lax.dot_general(
        Ep, Fhat, (((1,), (0,)), ((), ())),
        precision=_HI, preferred_element_type=jnp.float32)
    aq = jnp.abs(Q)

    Ftot = jnp.sum(Fc, axis=1)
    T = Ep * Ftot[None, :]
    z2 = []
    for h in range(H):
        zsum = jnp.sum(aq[:, h * N:(h + 1) * N], axis=1, keepdims=True)
        z2.append(T[:, 2 * h:2 * h + 1] + T[:, 2 * h + 1:2 * h + 2] + zsum)
    Z2 = jnp.concatenate(z2, axis=1)
    Cc = (1.0 / H) / Z2

    Cc2 = jnp.concatenate([Cc[:, :, None], Cc[:, :, None]], axis=2)
    G = Ep * Cc2.reshape(R, 2 * H)
    M = jax.lax.dot_general(
        G, Fc, (((1,), (0,)), ((), ())),
        precision=_HI, preferred_element_type=jnp.float32)

    acc = M
    for h in range(H):
        acc = acc + Cc[:, h:h + 1] * aq[:, h * N:(h + 1) * N]
    out_ref[0] = acc


@functools.partial(jax.jit, static_argnames=())
def kernel(x, W, a_src, a_dst):
    vec_shape = jax.ShapeDtypeStruct((B, N, H), jnp.float32)
    vecT_shape = jax.ShapeDtypeStruct((B, H, N), jnp.float32)
    E1, E2, F1, F2 = pl.pallas_call(
        _vec_kernel,
        grid=(B,),
        in_specs=[
            pl.BlockSpec((1, N, D), lambda b: (b, 0, 0)),
            pl.BlockSpec((H, D, F), lambda b: (0, 0, 0)),
            pl.BlockSpec((H, F), lambda b: (0, 0)),
            pl.BlockSpec((H, F), lambda b: (0, 0)),
        ],
        out_specs=[
            pl.BlockSpec((1, N, H), lambda b: (b, 0, 0)),
            pl.BlockSpec((1, N, H), lambda b: (b, 0, 0)),
            pl.BlockSpec((1, H, N), lambda b: (b, 0, 0)),
            pl.BlockSpec((1, H, N), lambda b: (b, 0, 0)),
        ],
        out_shape=[vec_shape, vec_shape, vecT_shape, vecT_shape],
    )(x, W, a_src, a_dst)

    Ep = jnp.stack([E1, E2], axis=3).reshape(B, N, 2 * H)
    Fpair = jnp.stack([F1, -F2], axis=2)
    eye = jnp.eye(H, dtype=jnp.float32)
    Fhat = (Fpair[:, :, :, None, :] * eye[None, :, None, :, None]
            ).reshape(B, 2 * H, H * N)
    Fc = jnp.stack([F1, F2], axis=2).reshape(B, 2 * H, N)

    return pl.pallas_call(
        _adj_kernel,
        grid=(B, N // R),
        in_specs=[
            pl.BlockSpec((1, R, 2 * H), lambda b, i: (b, i, 0)),
            pl.BlockSpec((1, 2 * H, H * N), lambda b, i: (b, 0, 0)),
            pl.BlockSpec((1, 2 * H, N), lambda b, i: (b, 0, 0)),
        ],
        out_specs=pl.BlockSpec((1, R, N), lambda b, i: (b, i, 0)),
        out_shape=jax.ShapeDtypeStruct((B, N, N), jnp.float32),
        compiler_params=pltpu.CompilerParams(
            dimension_semantics=("parallel", "parallel")),
    )(Ep, Fhat, Fc)


# speedup vs baseline: 1.5359x; 1.5359x over previous
import functools

import jax
import jax.numpy as jnp
from jax.experimental import pallas as pl
from jax.experimental.pallas import tpu as pltpu

B, N, D, H, F = 4, 1024, 128, 8, 64
R = 256
K3 = 6 * H

_NEG_SLOPE = 0.2
_HI = jax.lax.Precision.HIGHEST


def _vec_kernel(x_ref, W_ref, asrc_ref, adst_ref,
                E1_ref, E2_ref, F1_ref, F2_ref):
    xb = x_ref[0]
    W = W_ref[...]
    u_src = jax.lax.dot_general(
        W, asrc_ref[...], (((2,), (1,)), ((0,), (0,))),
        precision=_HI, preferred_element_type=jnp.float32)
    u_dst = jax.lax.dot_general(
        W, adst_ref[...], (((2,), (1,)), ((0,), (0,))),
        precision=_HI, preferred_element_type=jnp.float32)
    s = jax.lax.dot_general(
        xb, u_src, (((1,), (1,)), ((), ())),
        precision=_HI, preferred_element_type=jnp.float32)
    dT = jax.lax.dot_general(
        u_dst, xb, (((1,), (1,)), ((), ())),
        precision=_HI, preferred_element_type=jnp.float32)
    dmax = jnp.max(dT, axis=1)
    sm = s + dmax[None, :]
    m = jnp.maximum(sm, _NEG_SLOPE * sm)
    E1_ref[0] = jnp.exp(s - m)
    E2_ref[0] = jnp.exp(_NEG_SLOPE * s - m)
    F1_ref[0] = jnp.exp(dT)
    F2_ref[0] = jnp.exp(_NEG_SLOPE * dT)


def _split3(v, axis):
    hi = v.astype(jnp.bfloat16)
    lo = (v - hi.astype(jnp.float32)).astype(jnp.bfloat16)
    return jnp.concatenate([hi, hi, lo], axis=axis)


def _split3_other(v, axis):
    hi = v.astype(jnp.bfloat16)
    lo = (v - hi.astype(jnp.float32)).astype(jnp.bfloat16)
    return jnp.concatenate([hi, lo, hi], axis=axis)


def _adj_kernel(Ep_ref, EpQ_ref, Fhat_ref, Fc_ref, out_ref):
    Ep = Ep_ref[0]
    EpQ = EpQ_ref[0]
    Fhat = Fhat_ref[0]
    Fc = Fc_ref[0]

    Q = jax.lax.dot_general(
        EpQ, Fhat, (((1,), (0,)), ((), ())),
        preferred_element_type=jnp.float32)
    aq = jnp.abs(Q)

    Ftot = jnp.sum(Fc, axis=1)
    T = Ep * Ftot[None, :]
    z2 = []
    for h in range(H):
        zsum = jnp.sum(aq[:, h * N:(h + 1) * N], axis=1, keepdims=True)
        z2.append(T[:, 2 * h:2 * h + 1] + T[:, 2 * h + 1:2 * h + 2] + zsum)
    Z2 = jnp.concatenate(z2, axis=1)
    Cc = (1.0 / H) / Z2

    Cc2 = jnp.concatenate([Cc[:, :, None], Cc[:, :, None]], axis=2)
    G = Ep * Cc2.reshape(R, 2 * H)
    G3 = _split3(G, axis=1)
    Fc3 = _split3_other(Fc, axis=0)
    M = jax.lax.dot_general(
        G3, Fc3, (((1,), (0,)), ((), ())),
        preferred_element_type=jnp.float32)

    acc = M
    for h in range(H):
        acc = acc + Cc[:, h:h + 1] * aq[:, h * N:(h + 1) * N]
    out_ref[0] = acc


@functools.partial(jax.jit, static_argnames=())
def kernel(x, W, a_src, a_dst):
    vec_shape = jax.ShapeDtypeStruct((B, N, H), jnp.float32)
    vecT_shape = jax.ShapeDtypeStruct((B, H, N), jnp.float32)
    E1, E2, F1, F2 = pl.pallas_call(
        _vec_kernel,
        grid=(B,),
        in_specs=[
            pl.BlockSpec((1, N, D), lambda b: (b, 0, 0)),
            pl.BlockSpec((H, D, F), lambda b: (0, 0, 0)),
            pl.BlockSpec((H, F), lambda b: (0, 0)),
            pl.BlockSpec((H, F), lambda b: (0, 0)),
        ],
        out_specs=[
            pl.BlockSpec((1, N, H), lambda b: (b, 0, 0)),
            pl.BlockSpec((1, N, H), lambda b: (b, 0, 0)),
            pl.BlockSpec((1, H, N), lambda b: (b, 0, 0)),
            pl.BlockSpec((1, H, N), lambda b: (b, 0, 0)),
        ],
        out_shape=[vec_shape, vec_shape, vecT_shape, vecT_shape],
    )(x, W, a_src, a_dst)

    f32 = jnp.float32
    bf16 = jnp.bfloat16

    Ep = jnp.stack([E1, E2], axis=3).reshape(B, N, 2 * H)

    def hl(v):
        hi = v.astype(bf16)
        lo = (v - hi.astype(f32)).astype(bf16)
        return hi, lo

    e1h, e1l = hl(E1)
    e2h, e2l = hl(E2)
    f1h, f1l = hl(F1)
    f2h, f2l = hl(F2)
    EpQ = jnp.stack([e1h, e1h, e1l, e2h, e2h, e2l],
                    axis=3).reshape(B, N, K3)
    Fq = jnp.stack([f1h, f1l, f1h, -f2h, -f2l, -f2h],
                   axis=2)
    eye = jnp.eye(H, dtype=bf16)
    Fhat = (Fq[:, :, :, None, :] * eye[None, :, None, :, None]
            ).reshape(B, K3, H * N)

    Fc = jnp.stack([F1, F2], axis=2).reshape(B, 2 * H, N)

    return pl.pallas_call(
        _adj_kernel,
        grid=(B, N // R),
        in_specs=[
            pl.BlockSpec((1, R, 2 * H), lambda b, i: (b, i, 0)),
            pl.BlockSpec((1, R, K3), lambda b, i: (b, i, 0)),
            pl.BlockSpec((1, K3, H * N), lambda b, i: (b, 0, 0)),
            pl.BlockSpec((1, 2 * H, N), lambda b, i: (b, 0, 0)),
        ],
        out_specs=pl.BlockSpec((1, R, N), lambda b, i: (b, i, 0)),
        out_shape=jax.ShapeDtypeStruct((B, N, N), jnp.float32),
        compiler_params=pltpu.CompilerParams(
            dimension_semantics=("parallel", "parallel")),
    )(Ep, EpQ, Fhat, Fc)


# speedup vs baseline: 3.2746x; 2.1321x over previous
import functools

import jax
import jax.numpy as jnp
from jax.experimental import pallas as pl
from jax.experimental.pallas import tpu as pltpu

B, N, D, H, F = 4, 1024, 128, 8, 64
R = 256

_NEG_SLOPE = 0.2
_HI = jax.lax.Precision.HIGHEST


def _adj_kernel(x_ref, W_ref, asrc_ref, adst_ref, out_ref,
                E1_scr, E2_scr, F1_scr, F2_scr):
    i = pl.program_id(1)

    @pl.when(i == 0)
    def _compute_factors():
        xb = x_ref[0]
        W = W_ref[...]
        u_src = jax.lax.dot_general(
            W, asrc_ref[...], (((2,), (1,)), ((0,), (0,))),
            precision=_HI, preferred_element_type=jnp.float32)
        u_dst = jax.lax.dot_general(
            W, adst_ref[...], (((2,), (1,)), ((0,), (0,))),
            precision=_HI, preferred_element_type=jnp.float32)
        s = jax.lax.dot_general(
            xb, u_src, (((1,), (1,)), ((), ())),
            precision=_HI, preferred_element_type=jnp.float32)
        dT = jax.lax.dot_general(
            u_dst, xb, (((1,), (1,)), ((), ())),
            precision=_HI, preferred_element_type=jnp.float32)
        dmax = jnp.max(dT, axis=1)
        sm = s + dmax[None, :]
        m = jnp.maximum(sm, _NEG_SLOPE * sm)
        E1_scr[...] = jnp.exp(s - m)
        E2_scr[...] = jnp.exp(_NEG_SLOPE * s - m)
        F1_scr[...] = jnp.exp(dT)
        F2_scr[...] = jnp.exp(_NEG_SLOPE * dT)

    rows = pl.ds(i * R, R)
    acc = jnp.zeros((R, N), jnp.float32)
    for h in range(H):
        e1 = E1_scr[rows, h:h + 1]
        e2 = E2_scr[rows, h:h + 1]
        f1 = F1_scr[h:h + 1, :]
        f2 = F2_scr[h:h + 1, :]
        p = jnp.maximum(e1 * f1, e2 * f2)
        z = jnp.sum(p, axis=1, keepdims=True)
        acc = acc + p * ((1.0 / H) / z)
    out_ref[0] = acc


@functools.partial(jax.jit, static_argnames=())
def kernel(x, W, a_src, a_dst):
    return pl.pallas_call(
        _adj_kernel,
        grid=(B, N // R),
        in_specs=[
            pl.BlockSpec((1, N, D), lambda b, i: (b, 0, 0)),
            pl.BlockSpec((H, D, F), lambda b, i: (0, 0, 0)),
            pl.BlockSpec((H, F), lambda b, i: (0, 0)),
            pl.BlockSpec((H, F), lambda b, i: (0, 0)),
        ],
        out_specs=pl.BlockSpec((1, R, N), lambda b, i: (b, i, 0)),
        out_shape=jax.ShapeDtypeStruct((B, N, N), jnp.float32),
        scratch_shapes=[
            pltpu.VMEM((N, H), jnp.float32),
            pltpu.VMEM((N, H), jnp.float32),
            pltpu.VMEM((H, N), jnp.float32),
            pltpu.VMEM((H, N), jnp.float32),
        ],
        compiler_params=pltpu.CompilerParams(
            dimension_semantics=("arbitrary", "arbitrary")),
    )(x, W, a_src, a_dst)


# speedup vs baseline: 3.5248x; 1.0764x over previous
import functools

import jax
import jax.numpy as jnp
from jax.experimental import pallas as pl
from jax.experimental.pallas import tpu as pltpu

B, N, D, H, F = 4, 1024, 128, 8, 64
R = 512

_NEG_SLOPE = 0.2
_HI = jax.lax.Precision.HIGHEST


def _adj_kernel(x_ref, W_ref, asrc_ref, adst_ref, out_ref,
                E1_scr, E2_scr, F1_scr, F2_scr):
    i = pl.program_id(1)

    @pl.when(i == 0)
    def _compute_factors():
        xb = x_ref[0]
        W = W_ref[...]
        u_src = jax.lax.dot_general(
            W, asrc_ref[...], (((2,), (1,)), ((0,), (0,))),
            precision=_HI, preferred_element_type=jnp.float32)
        u_dst = jax.lax.dot_general(
            W, adst_ref[...], (((2,), (1,)), ((0,), (0,))),
            precision=_HI, preferred_element_type=jnp.float32)
        s = jax.lax.dot_general(
            xb, u_src, (((1,), (1,)), ((), ())),
            precision=_HI, preferred_element_type=jnp.float32)
        dT = jax.lax.dot_general(
            u_dst, xb, (((1,), (1,)), ((), ())),
            precision=_HI, preferred_element_type=jnp.float32)
        dmax = jnp.max(dT, axis=1)
        sm = s + dmax[None, :]
        m = jnp.maximum(sm, _NEG_SLOPE * sm)
        E1_scr[...] = jnp.exp(s - m)
        E2_scr[...] = jnp.exp(_NEG_SLOPE * s - m)
        F1_scr[...] = jnp.exp(dT)
        F2_scr[...] = jnp.exp(_NEG_SLOPE * dT)

    rows = pl.ds(i * R, R)
    acc = jnp.zeros((R, N), jnp.float32)
    for h in range(H):
        e1 = E1_scr[rows, h:h + 1]
        e2 = E2_scr[rows, h:h + 1]
        f1 = F1_scr[h:h + 1, :]
        f2 = F2_scr[h:h + 1, :]
        p = jnp.maximum(e1 * f1, e2 * f2)
        z = jnp.sum(p, axis=1, keepdims=True)
        acc = acc + p * ((1.0 / H) / z)
    out_ref[0] = acc


@functools.partial(jax.jit, static_argnames=())
def kernel(x, W, a_src, a_dst):
    return pl.pallas_call(
        _adj_kernel,
        grid=(B, N // R),
        in_specs=[
            pl.BlockSpec((1, N, D), lambda b, i: (b, 0, 0)),
            pl.BlockSpec((H, D, F), lambda b, i: (0, 0, 0)),
            pl.BlockSpec((H, F), lambda b, i: (0, 0)),
            pl.BlockSpec((H, F), lambda b, i: (0, 0)),
        ],
        out_specs=pl.BlockSpec((1, R, N), lambda b, i: (b, i, 0)),
        out_shape=jax.ShapeDtypeStruct((B, N, N), jnp.float32),
        scratch_shapes=[
            pltpu.VMEM((N, H), jnp.float32),
            pltpu.VMEM((N, H), jnp.float32),
            pltpu.VMEM((H, N), jnp.float32),
            pltpu.VMEM((H, N), jnp.float32),
        ],
        compiler_params=pltpu.CompilerParams(
            dimension_semantics=("arbitrary", "arbitrary")),
    )(x, W, a_src, a_dst)


# speedup vs baseline: 4.3900x; 1.2455x over previous
import functools

import jax
import jax.numpy as jnp
from jax.experimental import pallas as pl
from jax.experimental.pallas import tpu as pltpu

B, N, D, H, F = 4, 1024, 128, 8, 64
R = 1024

_NEG_SLOPE = 0.2
_HI = jax.lax.Precision.HIGHEST


def _adj_kernel(x_ref, W_ref, asrc_ref, adst_ref, out_ref,
                RHO_scr, F1_scr, F2_scr):
    i = pl.program_id(1)

    @pl.when(i == 0)
    def _compute_factors():
        xb = x_ref[0]
        W = W_ref[...]
        u_src = jax.lax.dot_general(
            W, asrc_ref[...], (((2,), (1,)), ((0,), (0,))),
            precision=_HI, preferred_element_type=jnp.float32)
        u_dst = jax.lax.dot_general(
            W, adst_ref[...], (((2,), (1,)), ((0,), (0,))),
            precision=_HI, preferred_element_type=jnp.float32)
        s = jax.lax.dot_general(
            xb, u_src, (((1,), (1,)), ((), ())),
            precision=_HI, preferred_element_type=jnp.float32)
        dT = jax.lax.dot_general(
            u_dst, xb, (((1,), (1,)), ((), ())),
            precision=_HI, preferred_element_type=jnp.float32)
        RHO_scr[...] = jnp.exp((1.0 - _NEG_SLOPE) * s)
        F1_scr[...] = jnp.exp(dT)
        F2_scr[...] = jnp.exp(_NEG_SLOPE * dT)

    rows = pl.ds(i * R, R)
    acc = jnp.zeros((R, N), jnp.float32)
    for h in range(H):
        rho = RHO_scr[rows, h:h + 1]
        f1 = F1_scr[h:h + 1, :]
        f2 = F2_scr[h:h + 1, :]
        w = jnp.maximum(rho * f1, f2)
        z = jnp.sum(w, axis=1, keepdims=True)
        acc = acc + w * ((1.0 / H) / z)
    out_ref[0] = acc


@functools.partial(jax.jit, static_argnames=())
def kernel(x, W, a_src, a_dst):
    return pl.pallas_call(
        _adj_kernel,
        grid=(B, N // R),
        in_specs=[
            pl.BlockSpec((1, N, D), lambda b, i: (b, 0, 0)),
            pl.BlockSpec((H, D, F), lambda b, i: (0, 0, 0)),
            pl.BlockSpec((H, F), lambda b, i: (0, 0)),
            pl.BlockSpec((H, F), lambda b, i: (0, 0)),
        ],
        out_specs=pl.BlockSpec((1, R, N), lambda b, i: (b, i, 0)),
        out_shape=jax.ShapeDtypeStruct((B, N, N), jnp.float32),
        scratch_shapes=[
            pltpu.VMEM((N, H), jnp.float32),
            pltpu.VMEM((H, N), jnp.float32),
            pltpu.VMEM((H, N), jnp.float32),
        ],
        compiler_params=pltpu.CompilerParams(
            dimension_semantics=("arbitrary", "arbitrary")),
    )(x, W, a_src, a_dst)


# speedup vs baseline: 4.5344x; 1.0329x over previous
import functools

import jax
import jax.numpy as jnp
from jax.experimental import pallas as pl
from jax.experimental.pallas import tpu as pltpu

B, N, D, H, F = 4, 1024, 128, 8, 64
R = 1024

_NEG_SLOPE = 0.2
_HI = jax.lax.Precision.HIGHEST


def _adj_kernel(x_ref, W_ref, asrc_ref, adst_ref, out_ref,
                RHO_scr, F1_scr, F2_scr):
    b = pl.program_id(0)

    @pl.when(b == 0)
    def _compute_factors():
        xall = x_ref[...].reshape(B * N, D)
        W = W_ref[...]
        u_src = jax.lax.dot_general(
            W, asrc_ref[...], (((2,), (1,)), ((0,), (0,))),
            precision=_HI, preferred_element_type=jnp.float32)
        u_dst = jax.lax.dot_general(
            W, adst_ref[...], (((2,), (1,)), ((0,), (0,))),
            precision=_HI, preferred_element_type=jnp.float32)
        s = jax.lax.dot_general(
            xall, u_src, (((1,), (1,)), ((), ())),
            precision=_HI, preferred_element_type=jnp.float32)
        dT = jax.lax.dot_general(
            u_dst, xall, (((1,), (1,)), ((), ())),
            precision=_HI, preferred_element_type=jnp.float32)
        RHO_scr[...] = jnp.exp((1.0 - _NEG_SLOPE) * s)
        F1_scr[...] = jnp.exp(dT)
        F2_scr[...] = jnp.exp(_NEG_SLOPE * dT)

    rows = pl.ds(b * N, R)
    cols = pl.ds(b * N, N)
    acc = jnp.zeros((R, N), jnp.float32)
    for h in range(H):
        rho = RHO_scr[rows, h:h + 1]
        f1 = F1_scr[h:h + 1, cols]
        f2 = F2_scr[h:h + 1, cols]
        w = jnp.maximum(rho * f1, f2)
        z = jnp.sum(w, axis=1, keepdims=True)
        acc = acc + w * ((1.0 / H) / z)
    out_ref[0] = acc


@functools.partial(jax.jit, static_argnames=())
def kernel(x, W, a_src, a_dst):
    return pl.pallas_call(
        _adj_kernel,
        grid=(B, N // R),
        in_specs=[
            pl.BlockSpec((B, N, D), lambda b, i: (0, 0, 0)),
            pl.BlockSpec((H, D, F), lambda b, i: (0, 0, 0)),
            pl.BlockSpec((H, F), lambda b, i: (0, 0)),
            pl.BlockSpec((H, F), lambda b, i: (0, 0)),
        ],
        out_specs=pl.BlockSpec((1, R, N), lambda b, i: (b, i, 0)),
        out_shape=jax.ShapeDtypeStruct((B, N, N), jnp.float32),
        scratch_shapes=[
            pltpu.VMEM((B * N, H), jnp.float32),
            pltpu.VMEM((H, B * N), jnp.float32),
            pltpu.VMEM((H, B * N), jnp.float32),
        ],
        compiler_params=pltpu.CompilerParams(
            dimension_semantics=("arbitrary", "arbitrary")),
    )(x, W, a_src, a_dst)
